# baseline (speedup 1.0000x reference)
import functools

import jax
import jax.numpy as jnp
from jax import lax
from jax.experimental import pallas as pl
from jax.experimental.pallas import tpu as pltpu
from jax.experimental.pallas import tpu_sc as plsc

N = 18432
K = 8192
D = 256
BN = 256
NB = N // BN
COMMIT = 0.25


def _argmin_body(x_ref, w_ref, idx_ref, s2_ref):
    i = pl.program_id(0)
    w = w_ref[...]

    @pl.when(i == 0)
    def _():
        s2_ref[...] = jnp.sum(w * w, axis=1).reshape(1, K)

    x = x_ref[...]
    s1 = jnp.sum(x * x, axis=1).reshape(BN, 1)
    mm = lax.dot_general(
        x, w, (((1,), (1,)), ((), ())), preferred_element_type=jnp.float32
    )
    dist = (s1 + s2_ref[...]) - 2.0 * mm
    m = jnp.min(dist, axis=1, keepdims=True)
    iota = lax.broadcasted_iota(jnp.int32, (BN, K), 1)
    cand = jnp.where(dist == m, iota, K)
    idx_ref[...] = jnp.min(cand, axis=1).reshape(1, 1, BN)


def _tc_argmin(inputs, weight):
    return pl.pallas_call(
        _argmin_body,
        grid=(NB,),
        in_specs=[
            pl.BlockSpec((BN, D), lambda i: (i, 0)),
            pl.BlockSpec((K, D), lambda i: (0, 0)),
        ],
        out_specs=pl.BlockSpec((1, 1, BN), lambda i: (i, 0, 0)),
        out_shape=jax.ShapeDtypeStruct((NB, 1, BN), jnp.int32),
        scratch_shapes=[pltpu.VMEM((1, K), jnp.float32)],
    )(inputs, weight)


_NC = 2
_NS = 16
_NW = _NC * _NS
_BPW = N // _NW
_CHUNK = 96
_NCHUNK = _BPW // _CHUNK


def _sc_gather_body(table_hbm, idx_hbm, out_hbm, idx_v, rows_v, sem):
    wid = lax.axis_index("s") * _NC + lax.axis_index("c")
    base = wid * _BPW
    for c in range(_NCHUNK):
        pltpu.sync_copy(idx_hbm.at[pl.ds(base + c * _CHUNK, _CHUNK)], idx_v.at[c])
    for c in range(_NCHUNK):
        pltpu.async_copy(table_hbm.at[idx_v.at[c]], rows_v, sem).wait()
        pltpu.sync_copy(rows_v, out_hbm.at[pl.ds(base + c * _CHUNK, _CHUNK)])


@functools.cache
def _sc_gather():
    return pl.kernel(
        _sc_gather_body,
        mesh=plsc.VectorSubcoreMesh(core_axis_name="c", subcore_axis_name="s"),
        out_type=jax.ShapeDtypeStruct((N, D), jnp.float32),
        scratch_types=[
            pltpu.VMEM((_NCHUNK, _CHUNK), jnp.int32),
            pltpu.VMEM((_CHUNK, D), jnp.float32),
            pltpu.SemaphoreType.DMA,
        ],
    )


def _finish_body(x_ref, q_ref, qst_ref, loss_ref, acc_ref):
    i = pl.program_id(0)
    x = x_ref[...]
    q = q_ref[...]
    qst_ref[...] = x + (q - x)
    d = q - x
    part = jnp.sum(d * d)

    @pl.when(i == 0)
    def _():
        acc_ref[0, 0] = part

    @pl.when(i > 0)
    def _():
        acc_ref[0, 0] = acc_ref[0, 0] + part

    @pl.when(i == NB - 1)
    def _():
        loss_ref[0, 0] = acc_ref[0, 0] * ((1.0 + COMMIT) / (N * D))


def _tc_finish(inputs, quant):
    return pl.pallas_call(
        _finish_body,
        grid=(NB,),
        in_specs=[
            pl.BlockSpec((BN, D), lambda i: (i, 0)),
            pl.BlockSpec((BN, D), lambda i: (i, 0)),
        ],
        out_specs=[
            pl.BlockSpec((BN, D), lambda i: (i, 0)),
            pl.BlockSpec(memory_space=pltpu.SMEM),
        ],
        out_shape=[
            jax.ShapeDtypeStruct((N, D), jnp.float32),
            jax.ShapeDtypeStruct((1, 1), jnp.float32),
        ],
        scratch_shapes=[pltpu.SMEM((1, 1), jnp.float32)],
    )(inputs, quant)


def kernel(inputs, weight):
    flat = inputs.reshape(N, D)
    idx = _tc_argmin(flat, weight).reshape(N)
    quant = _sc_gather()(weight, idx)
    qst, loss = _tc_finish(flat, quant)
    return qst, loss[0, 0]
